# Initial kernel scaffold; baseline (speedup 1.0000x reference)
import jax
import jax.numpy as jnp
from jax.experimental import pallas as pl


def kernel(X, W):
    raise NotImplementedError("write your pallas kernel here")


# speedup vs baseline: 1.5052x; 1.5052x over previous
import functools

import jax
import jax.numpy as jnp
from jax import lax
from jax.experimental import pallas as pl
from jax.experimental.pallas import tpu as pltpu
from jax.experimental.pallas import tpu_sc as plsc

OUT_DIM = 32
N_IDX = 5 * 4096 * 26
NUM_WORKERS = 32
PER_WORKER = N_IDX // NUM_WORKERS
CHUNK = 1664
NCHUNKS = PER_WORKER // CHUNK

_mesh = plsc.VectorSubcoreMesh(core_axis_name="c", subcore_axis_name="s")


@functools.partial(
    pl.kernel,
    mesh=_mesh,
    compiler_params=pltpu.CompilerParams(use_tc_tiling_on_sc=False),
    out_type=jax.ShapeDtypeStruct((N_IDX, OUT_DIM), jnp.float32),
    scratch_types=[
        pltpu.VMEM((CHUNK,), jnp.int32),
        pltpu.VMEM((CHUNK, OUT_DIM), jnp.float32),
        pltpu.SemaphoreType.DMA,
    ],
)
def _gather_sc(idx_hbm, table_hbm, out_hbm, idx_v, rows_v, sem):
    wid = lax.axis_index("s") * 2 + lax.axis_index("c")
    base = wid * PER_WORKER

    def body(c, carry):
        off = base + c * CHUNK
        pltpu.sync_copy(idx_hbm.at[pl.ds(off, CHUNK)], idx_v)
        pltpu.async_copy(table_hbm.at[idx_v], rows_v, sem).wait()
        pltpu.sync_copy(rows_v, out_hbm.at[pl.ds(off, CHUNK)])
        return carry

    lax.fori_loop(0, NCHUNKS, body, 0)


_PEN_ROWS = 250000
_PEN_BLOCK = 10000


def _penalty_body(w_ref, out_ref):
    i = pl.program_id(0)
    blk = w_ref[...]
    s = 0.5 * jnp.sum(blk * blk) + jnp.sum(jnp.abs(blk))

    @pl.when(i == 0)
    def _():
        out_ref[0, 0] = 0.0

    out_ref[0, 0] += s


def _penalty_tc(w2d):
    return pl.pallas_call(
        _penalty_body,
        grid=(_PEN_ROWS // _PEN_BLOCK,),
        in_specs=[pl.BlockSpec((_PEN_BLOCK, 128), lambda i: (i, 0))],
        out_specs=pl.BlockSpec(memory_space=pltpu.SMEM),
        out_shape=jax.ShapeDtypeStruct((1, 1), jnp.float32),
    )(w2d)


def kernel(X, W):
    n_samples, n_batch, input_dim = X.shape
    idx = X.reshape(-1)
    rows = _gather_sc(idx, W)
    net = rows.reshape(n_samples, n_batch, input_dim * OUT_DIM)
    pen = _penalty_tc(W.reshape(_PEN_ROWS, 128))[0, 0]
    return net, pen


# speedup vs baseline: 2.1624x; 1.4366x over previous
import functools

import jax
import jax.numpy as jnp
from jax import lax
from jax.experimental import pallas as pl
from jax.experimental.pallas import tpu as pltpu
from jax.experimental.pallas import tpu_sc as plsc

OUT_DIM = 32
N_IDX = 5 * 4096 * 26
NUM_WORKERS = 32
PER_WORKER = N_IDX // NUM_WORKERS
CHUNK = 1664
NCHUNKS = PER_WORKER // CHUNK

_mesh = plsc.VectorSubcoreMesh(core_axis_name="c", subcore_axis_name="s")


@functools.partial(
    pl.kernel,
    mesh=_mesh,
    compiler_params=pltpu.CompilerParams(use_tc_tiling_on_sc=False),
    out_type=jax.ShapeDtypeStruct((N_IDX, OUT_DIM), jnp.float32),
    scratch_types=[
        pltpu.VMEM((CHUNK,), jnp.int32),
        pltpu.VMEM((CHUNK, OUT_DIM), jnp.float32),
        pltpu.SemaphoreType.DMA,
    ],
)
def _gather_sc(idx_hbm, table_hbm, out_hbm, idx_v, rows_v, sem):
    wid = lax.axis_index("s") * 2 + lax.axis_index("c")
    base = wid * PER_WORKER

    def body(c, carry):
        off = base + c * CHUNK
        pltpu.sync_copy(idx_hbm.at[pl.ds(off, CHUNK)], idx_v)
        pltpu.async_copy(table_hbm.at[idx_v], rows_v, sem).wait()
        pltpu.sync_copy(rows_v, out_hbm.at[pl.ds(off, CHUNK)])
        return carry

    lax.fori_loop(0, NCHUNKS, body, 0)


_PEN_BLOCK = 65536


def _penalty_body(ncat, w_ref, out_ref):
    i = pl.program_id(0)
    blk = w_ref[...]
    col = jax.lax.broadcasted_iota(jnp.int32, blk.shape, 1) + i * _PEN_BLOCK
    v = jnp.where(col < ncat, blk, 0.0)
    s = 0.5 * jnp.sum(v * v) + jnp.sum(jnp.abs(v))

    @pl.when(i == 0)
    def _():
        out_ref[0, 0] = 0.0

    out_ref[0, 0] += s


def _penalty_tc(wt):
    d, ncat = wt.shape
    nblk = (ncat + _PEN_BLOCK - 1) // _PEN_BLOCK
    return pl.pallas_call(
        functools.partial(_penalty_body, ncat),
        grid=(nblk,),
        in_specs=[pl.BlockSpec((d, _PEN_BLOCK), lambda i: (0, i))],
        out_specs=pl.BlockSpec(memory_space=pltpu.SMEM),
        out_shape=jax.ShapeDtypeStruct((1, 1), jnp.float32),
    )(wt)


def kernel(X, W):
    n_samples, n_batch, input_dim = X.shape
    idx = X.reshape(-1)
    rows = _gather_sc(idx, W)
    net = rows.reshape(n_samples, n_batch, input_dim * OUT_DIM)
    pen = _penalty_tc(W.T)[0, 0]
    return net, pen
